# baseline (speedup 1.0000x reference)
import jax
import jax.numpy as jnp
from jax.experimental import pallas as pl
from jax.experimental.pallas import tpu as pltpu

_BB = 8
_NS = 4


def _gcn_body(x1_ref, x2_ref, adj_ref, w_ref, b_ref, out_ref):
    adj = adj_ref[...]
    C = adj.shape[0]
    deg_r = jnp.sum(adj, axis=1, keepdims=True) + 1.0
    deg_c = jnp.sum(adj, axis=0, keepdims=True) + 1.0
    dinv_r = jax.lax.rsqrt(deg_r)
    dinv_c = jax.lax.rsqrt(deg_c)
    eye = jnp.eye(C, dtype=adj.dtype)
    A = (adj + eye) * dinv_r * dinv_c * w_ref[0, 0]
    bias = b_ref[0, 0]
    dn = (((1,), (0,)), ((), ()))
    for j in range(_BB):
        out_ref[0, j] = jax.lax.dot_general(
            A, x1_ref[j], dn, preferred_element_type=jnp.float32) + bias
        out_ref[1, j] = jax.lax.dot_general(
            A, x2_ref[j], dn, preferred_element_type=jnp.float32) + bias


def kernel(x, adj, W, b):
    B, C, T = x.shape
    h = B // 2
    out = pl.pallas_call(
        _gcn_body,
        grid=(_NS,),
        in_specs=[
            pl.BlockSpec((_BB, C, T), lambda i: (i, 0, 0)),
            pl.BlockSpec((_BB, C, T), lambda i: (i + _NS, 0, 0)),
            pl.BlockSpec((C, C), lambda i: (0, 0)),
            pl.BlockSpec((1, 1), lambda i: (0, 0)),
            pl.BlockSpec((1, 1), lambda i: (0, 0)),
        ],
        out_specs=pl.BlockSpec((2, _BB, C, T), lambda i: (0, i, 0, 0)),
        out_shape=jax.ShapeDtypeStruct((2, h, C, T), jnp.float32),
    )(x, x, adj, W, b.reshape(1, 1))
    return out.reshape(B, C, T)


# speedup vs baseline: 1.1808x; 1.1808x over previous
import jax
import jax.numpy as jnp
from jax.experimental import pallas as pl
from jax.experimental.pallas import tpu as pltpu

_BB = 16
_NS = 2


def _gcn_body(x1_ref, x2_ref, adj_ref, w_ref, b_ref, out_ref):
    adj = adj_ref[...]
    C = adj.shape[0]
    deg_r = jnp.sum(adj, axis=1, keepdims=True) + 1.0
    deg_c = jnp.sum(adj, axis=0, keepdims=True) + 1.0
    dinv_r = jax.lax.rsqrt(deg_r)
    dinv_c = jax.lax.rsqrt(deg_c)
    eye = jnp.eye(C, dtype=adj.dtype)
    A = (adj + eye) * dinv_r * dinv_c * w_ref[0, 0]
    bias = b_ref[0, 0]
    dn = (((1,), (0,)), ((), ()))
    for j in range(_BB):
        out_ref[0, j] = jax.lax.dot_general(
            A, x1_ref[j], dn, preferred_element_type=jnp.float32) + bias
        out_ref[1, j] = jax.lax.dot_general(
            A, x2_ref[j], dn, preferred_element_type=jnp.float32) + bias


def kernel(x, adj, W, b):
    B, C, T = x.shape
    h = B // 2
    out = pl.pallas_call(
        _gcn_body,
        grid=(_NS,),
        in_specs=[
            pl.BlockSpec((_BB, C, T), lambda i: (i, 0, 0)),
            pl.BlockSpec((_BB, C, T), lambda i: (i + _NS, 0, 0)),
            pl.BlockSpec((C, C), lambda i: (0, 0)),
            pl.BlockSpec((1, 1), lambda i: (0, 0)),
            pl.BlockSpec((1, 1), lambda i: (0, 0)),
        ],
        out_specs=pl.BlockSpec((2, _BB, C, T), lambda i: (0, i, 0, 0)),
        out_shape=jax.ShapeDtypeStruct((2, h, C, T), jnp.float32),
    )(x, x, adj, W, b.reshape(1, 1))
    return out.reshape(B, C, T)


# speedup vs baseline: 1.1987x; 1.0152x over previous
import jax
import jax.numpy as jnp
from jax.experimental import pallas as pl

_BB = 32


def _gcn_body(x_ref, adj_ref, w_ref, b_ref, out_ref):
    adj = adj_ref[...]
    C = adj.shape[0]
    deg_r = jnp.sum(adj, axis=1, keepdims=True) + 1.0
    deg_c = jnp.sum(adj, axis=0, keepdims=True) + 1.0
    dinv_r = jax.lax.rsqrt(deg_r)
    dinv_c = jax.lax.rsqrt(deg_c)
    eye = jnp.eye(C, dtype=adj.dtype)
    A = (adj + eye) * dinv_r * dinv_c * w_ref[0, 0]
    bias = b_ref[0, 0]
    for i in range(x_ref.shape[0]):
        out_ref[i, :, :] = jax.lax.dot_general(
            A, x_ref[i], (((1,), (0,)), ((), ())),
            precision=jax.lax.Precision.DEFAULT,
            preferred_element_type=jnp.float32) + bias


def kernel(x, adj, W, b):
    B, C, T = x.shape
    out = pl.pallas_call(
        _gcn_body,
        grid=(B // _BB,),
        in_specs=[
            pl.BlockSpec((_BB, C, T), lambda i: (i, 0, 0)),
            pl.BlockSpec((C, C), lambda i: (0, 0)),
            pl.BlockSpec((1, 1), lambda i: (0, 0)),
            pl.BlockSpec((1, 1), lambda i: (0, 0)),
        ],
        out_specs=pl.BlockSpec((_BB, C, T), lambda i: (i, 0, 0)),
        out_shape=jax.ShapeDtypeStruct((B, C, T), jnp.float32),
    )(x, adj, W, b.reshape(1, 1))
    return out
